# baseline (speedup 1.0000x reference)
import jax
import jax.numpy as jnp
from jax.experimental import pallas as pl

_ANCHORS_W = (116.0, 156.0, 373.0)
_ANCHORS_H = (90.0, 198.0, 326.0)
_G = 76
_B = 16
_C = 255
_STRIDE = 8.0
_HC = 4
_GC = _HC * _G
_MR = _GC * _B


def _half_consts(base, gx8, gy8):
    lane = jax.lax.broadcasted_iota(jnp.int32, (_GC, 128), 1) + base
    k = lane % 85
    aw = jnp.where(lane < 85, _ANCHORS_W[0],
                   jnp.where(lane < 170, _ANCHORS_W[1], _ANCHORS_W[2]))
    ah = jnp.where(lane < 85, _ANCHORS_H[0],
                   jnp.where(lane < 170, _ANCHORS_H[1], _ANCHORS_H[2]))
    anch = jnp.where(k == 2, aw, ah).astype(jnp.float32)
    a_val = jnp.where(k == 0, _STRIDE + gx8,
                      jnp.where(k == 1, _STRIDE + gy8, 1.0))
    b_val = jnp.where(k < 2, _STRIDE, 1.0).astype(jnp.float32)
    is_wh = (k == 2) | (k == 3)
    return a_val.astype(jnp.float32), b_val, anch, is_wh


def _decode_body(x0_ref, x1_ref, o_ref):
    j = pl.program_id(0)

    row = jax.lax.broadcasted_iota(jnp.int32, (_GC, 128), 0)
    gx8 = (row % _G).astype(jnp.float32) * _STRIDE
    gy8 = (j * _HC + row // _G).astype(jnp.float32) * _STRIDE

    a0, b0, anch0, wh0 = _half_consts(0, gx8, gy8)
    a1, b1, anch1, wh1 = _half_consts(128, gx8, gy8)

    def decode(u, a_val, b_val, anch, is_wh):
        e = jnp.exp(u)
        inv = 1.0 / (1.0 + e)
        return jnp.where(is_wh, anch * e, a_val - b_val * inv)

    for b in range(_B):
        rows = pl.Slice(b, _GC, _B)
        v0 = decode(x0_ref[rows, :], a0, b0, anch0, wh0)
        v1 = decode(x1_ref[rows, :], a1, b1, anch1, wh1)
        o_ref[b, pl.Slice(0, _GC, 3), :] = v0[:, 0:85]
        o_ref[b, pl.Slice(1, _GC, 3), :] = jnp.concatenate(
            [v0[:, 85:128], v1[:, 0:42]], axis=1)
        o_ref[b, pl.Slice(2, _GC, 3), :] = v1[:, 42:127]


@jax.jit
def kernel(x):
    xt = jnp.transpose(x, (2, 3, 0, 1))
    xm = xt.reshape(_G * _G * _B, _C)
    out = pl.pallas_call(
        _decode_body,
        grid=(_G // _HC,),
        in_specs=[pl.BlockSpec((_MR, 128), lambda j: (j, 0)),
                  pl.BlockSpec((_MR, 128), lambda j: (j, 1))],
        out_specs=pl.BlockSpec((_B, 3 * _GC, 85), lambda j: (0, j, 0)),
        out_shape=jax.ShapeDtypeStruct((_B, _G * _G * 3, 85), x.dtype),
    )(xm, xm)
    return out
